# baseline (speedup 1.0000x reference)
import functools

import jax
import jax.numpy as jnp
from jax import lax
from jax.experimental import pallas as pl
from jax.experimental.pallas import tpu as pltpu
from jax.experimental.pallas import tpu_sc as plsc

MU2 = 7.0
D = 64
B = 16384
L = 16
NC = 2
NS = 16
NW = NC * NS
BW = B // NW
NG = BW // L


def _make_item_kernel():
  mesh = plsc.VectorSubcoreMesh(core_axis_name="c", subcore_axis_name="s")

  def body(i_idx_hbm, u_idx_hbm, i_emb_hbm, u_bias_hbm, i_bias_hbm,
           rows_hbm, part_hbm, i_idx_v, u_idx_v, rows_v, ub_v, ib_v,
           part_v, sem):
    wid = lax.axis_index("s") * NC + lax.axis_index("c")
    base = wid * BW

    pltpu.sync_copy(i_idx_hbm.at[pl.ds(base, BW)], i_idx_v)
    pltpu.sync_copy(u_idx_hbm.at[pl.ds(base, BW)], u_idx_v)

    c0 = pltpu.async_copy(i_emb_hbm.at[i_idx_v], rows_v, sem)
    c1 = pltpu.async_copy(u_bias_hbm.at[u_idx_v], ub_v, sem)
    c2 = pltpu.async_copy(i_bias_hbm.at[i_idx_v], ib_v, sem)
    c0.wait()
    c1.wait()
    c2.wait()

    def grp(g, carry):
      gbase = g * L
      part_v[pl.ds(gbase, L)] = (ub_v[pl.ds(gbase, L)] +
                                 ib_v[pl.ds(gbase, L)] + MU2)
      return carry

    lax.fori_loop(0, NG, grp, 0)
    pltpu.sync_copy(rows_v, rows_hbm.at[pl.ds(base, BW)])
    pltpu.sync_copy(part_v, part_hbm.at[pl.ds(base, BW)])

  return pl.kernel(
      body,
      out_type=(jax.ShapeDtypeStruct((B, D), jnp.float32),
                jax.ShapeDtypeStruct((B,), jnp.float32)),
      mesh=mesh,
      scratch_types=[
          pltpu.VMEM((BW,), jnp.int32),
          pltpu.VMEM((BW,), jnp.int32),
          pltpu.VMEM((BW, D), jnp.float32),
          pltpu.VMEM((BW,), jnp.float32),
          pltpu.VMEM((BW,), jnp.float32),
          pltpu.VMEM((BW,), jnp.float32),
          pltpu.SemaphoreType.DMA,
      ],
      compiler_params=pltpu.CompilerParams(
          needs_layout_passes=False, use_tc_tiling_on_sc=False,
          has_side_effects=pltpu.SideEffectType.PURE),
  )


def _make_user_kernel():
  mesh = plsc.VectorSubcoreMesh(core_axis_name="c", subcore_axis_name="s")

  def body(u_idx_hbm, u_emb_hbm, i_rows_hbm, part_hbm, out_hbm, u_idx_v,
           u_rows, i_rows, part_v, out_v, sem):
    wid = lax.axis_index("s") * NC + lax.axis_index("c")
    base = wid * BW

    pltpu.sync_copy(u_idx_hbm.at[pl.ds(base, BW)], u_idx_v)
    c0 = pltpu.async_copy(u_emb_hbm.at[u_idx_v], u_rows, sem)
    c1 = pltpu.async_copy(i_rows_hbm.at[pl.ds(base, BW)], i_rows, sem)
    c2 = pltpu.async_copy(part_hbm.at[pl.ds(base, BW)], part_v, sem)
    c0.wait()
    c1.wait()
    c2.wait()

    def grp(g, carry):
      gbase = g * L
      rows16 = gbase + lax.iota(jnp.int32, L)
      col = jnp.zeros((L,), jnp.int32)
      acc0 = part_v[pl.ds(gbase, L)]
      acc1 = jnp.zeros((L,), jnp.float32)
      acc2 = jnp.zeros((L,), jnp.float32)
      acc3 = jnp.zeros((L,), jnp.float32)
      accs = [acc0, acc1, acc2, acc3]
      for jd in range(D):
        ug = plsc.load_gather(u_rows, [rows16, col])
        vg = plsc.load_gather(i_rows, [rows16, col])
        accs[jd % 4] = accs[jd % 4] + ug * vg
        col = col + 1
      out_v[pl.ds(gbase, L)] = (accs[0] + accs[1]) + (accs[2] + accs[3])
      return carry

    lax.fori_loop(0, NG, grp, 0)
    pltpu.sync_copy(out_v, out_hbm.at[pl.ds(base, BW)])

  return pl.kernel(
      body,
      out_type=jax.ShapeDtypeStruct((B,), jnp.float32),
      mesh=mesh,
      scratch_types=[
          pltpu.VMEM((BW,), jnp.int32),
          pltpu.VMEM((BW, D), jnp.float32),
          pltpu.VMEM((BW, D), jnp.float32),
          pltpu.VMEM((BW,), jnp.float32),
          pltpu.VMEM((BW,), jnp.float32),
          pltpu.SemaphoreType.DMA,
      ],
      compiler_params=pltpu.CompilerParams(
          needs_layout_passes=False, use_tc_tiling_on_sc=False,
          has_side_effects=pltpu.SideEffectType.PURE),
  )


@jax.jit
def _mf(user_indices, item_indices, user_embedding, item_embedding,
        user_bias, item_bias):
  ub = user_bias.reshape(-1)
  ib = item_bias.reshape(-1)
  one = lax.optimization_barrier((user_indices[0] * 0 + 1).astype(jnp.float32))
  ue = user_embedding * one
  i_rows, part = _make_item_kernel()(item_indices, user_indices,
                                     item_embedding, ub, ib)
  return _make_user_kernel()(user_indices, ue, i_rows, part)


def kernel(user_indices, item_indices, user_embedding, item_embedding,
           user_bias, item_bias):
  return _mf(user_indices.astype(jnp.int32), item_indices.astype(jnp.int32),
             user_embedding, item_embedding, user_bias, item_bias)


# speedup vs baseline: 3.7174x; 3.7174x over previous
import functools

import jax
import jax.numpy as jnp
from jax import lax
from jax.experimental import pallas as pl
from jax.experimental.pallas import tpu as pltpu
from jax.experimental.pallas import tpu_sc as plsc

MU2 = 7.0
D = 64
B = 16384
NU = 1000000
L = 16
NC = 2
NS = 16
NW = NC * NS
BW = B // NW
NG = BW // L
RSW = BW + 8
PAD = 16


def _make_item_kernel():
  mesh = plsc.VectorSubcoreMesh(core_axis_name="c", subcore_axis_name="s")

  def body(i_idx_hbm, u_idx_hbm, i_emb_hbm, u_bias_hbm, i_bias_hbm,
           rows_hbm, part_hbm, i_idx_v, u_idx_v, rows_v, ub_v, ib_v,
           part_v, sem):
    wid = lax.axis_index("s") * NC + lax.axis_index("c")
    base = wid * BW

    pltpu.sync_copy(i_idx_hbm.at[pl.ds(base, BW)], i_idx_v)
    pltpu.sync_copy(u_idx_hbm.at[pl.ds(base, BW)], u_idx_v)

    c0 = pltpu.async_copy(i_emb_hbm.at[i_idx_v], rows_v, sem)
    c1 = pltpu.async_copy(u_bias_hbm.at[u_idx_v], ub_v, sem)
    c2 = pltpu.async_copy(i_bias_hbm.at[i_idx_v], ib_v, sem)
    c0.wait()
    c1.wait()
    c2.wait()

    def grp(g, carry):
      gbase = g * L
      part_v[pl.ds(gbase, L)] = (ub_v[pl.ds(gbase, L)] +
                                 ib_v[pl.ds(gbase, L)] + MU2)
      return carry

    lax.fori_loop(0, NG, grp, 0)
    pltpu.sync_copy(rows_v, rows_hbm.at[pl.ds(base, BW), pl.ds(0, D)])
    pltpu.sync_copy(part_v, part_hbm.at[pl.ds(base, BW)])

  return pl.kernel(
      body,
      out_type=(jax.ShapeDtypeStruct((B, 128), jnp.float32),
                jax.ShapeDtypeStruct((B,), jnp.float32)),
      mesh=mesh,
      scratch_types=[
          pltpu.VMEM((BW,), jnp.int32),
          pltpu.VMEM((BW,), jnp.int32),
          pltpu.VMEM((BW, D), jnp.float32),
          pltpu.VMEM((BW,), jnp.float32),
          pltpu.VMEM((BW,), jnp.float32),
          pltpu.VMEM((BW,), jnp.float32),
          pltpu.SemaphoreType.DMA,
      ],
      compiler_params=pltpu.CompilerParams(needs_layout_passes=False,
                                           use_tc_tiling_on_sc=False),
  )


def _make_user_kernel():
  mesh = plsc.VectorSubcoreMesh(core_axis_name="c", subcore_axis_name="s")

  def body(su_hbm, ord_hbm, sb_hbm, rs_hbm, nw_hbm, u_t_hbm, irows_hbm,
           dots_hbm, su_v, ord_v, sb_v, rs_v, nw_v, irows_v, slab_a,
           slab_b, out_v, sem, sem_a, sem_b):
    wid = lax.axis_index("s") * NC + lax.axis_index("c")
    base = wid * BW

    pltpu.sync_copy(su_hbm.at[pl.ds(base, BW)], su_v.at[pl.ds(0, BW)])
    pltpu.sync_copy(ord_hbm.at[pl.ds(base, BW)], ord_v)
    pltpu.sync_copy(sb_hbm.at[pl.ds(base, BW)], sb_v.at[pl.ds(0, BW)])
    pltpu.sync_copy(rs_hbm.at[pl.ds(wid * RSW, RSW)], rs_v.at[pl.ds(0, RSW)])
    pltpu.sync_copy(nw_hbm, nw_v.at[pl.ds(0, NW)])
    ci = pltpu.async_copy(irows_hbm.at[ord_v], irows_v, sem)

    def sread(ref, i):
      return ref[pl.ds(i, L)][0]

    n = sread(nw_v, wid)

    def fetch(k, buf, fsem):
      @pl.when(k < n)
      def _():
        sb = sread(sb_v, sread(rs_v, k))
        col = pl.multiple_of(sb, 128)
        pltpu.async_copy(u_t_hbm.at[:, pl.ds(col, 128)], buf, fsem)

    fetch(0, slab_a, sem_a)
    fetch(1, slab_b, sem_b)
    ci.wait()

    cvecs = [16 * q + lax.iota(jnp.int32, L) for q in range(4)]

    lane = lax.iota(jnp.int32, L)
    last_lane = lane == (L - 1)

    def run(k, buf):
      rs0 = sread(rs_v, k)
      rs1 = jnp.minimum(sread(rs_v, k + 1), BW)
      sb = sread(sb_v, rs0)

      def elem(e, carry):
        l = sread(su_v, e) - sb
        lv = jnp.full((L,), l, jnp.int32)
        acc = jnp.zeros((L,), jnp.float32)
        for q in range(4):
          uq = plsc.load_gather(buf, [cvecs[q], lv])
          iq = irows_v[e, pl.ds(16 * q, L)]
          acc = acc + uq * iq
        cs = plsc.cumsum(acc)
        plsc.store_scatter(out_v, [jnp.full((L,), e, jnp.int32)], cs,
                           mask=last_lane)
        return carry

      lax.fori_loop(rs0, rs1, elem, 0)

    def step(k, carry):
      parity = lax.rem(k, 2)

      @pl.when(parity == 0)
      def _():
        pltpu.make_async_copy(u_t_hbm.at[:, pl.ds(0, 128)], slab_a,
                              sem_a).wait()
        run(k, slab_a)
        fetch(k + 2, slab_a, sem_a)

      @pl.when(parity == 1)
      def _():
        pltpu.make_async_copy(u_t_hbm.at[:, pl.ds(0, 128)], slab_b,
                              sem_b).wait()
        run(k, slab_b)
        fetch(k + 2, slab_b, sem_b)

      return carry

    lax.fori_loop(0, n, step, 0)
    pltpu.sync_copy(out_v, dots_hbm.at[pl.ds(base, BW)])

  return pl.kernel(
      body,
      out_type=jax.ShapeDtypeStruct((B,), jnp.float32),
      mesh=mesh,
      scratch_types=[
          pltpu.VMEM((BW + PAD,), jnp.int32),
          pltpu.VMEM((BW,), jnp.int32),
          pltpu.VMEM((BW + PAD,), jnp.int32),
          pltpu.VMEM((RSW + PAD,), jnp.int32),
          pltpu.VMEM((NW + PAD,), jnp.int32),
          pltpu.VMEM((BW, 128), jnp.float32),
          pltpu.VMEM((D, 128), jnp.float32),
          pltpu.VMEM((D, 128), jnp.float32),
          pltpu.VMEM((BW,), jnp.float32),
          pltpu.SemaphoreType.DMA,
          pltpu.SemaphoreType.DMA,
          pltpu.SemaphoreType.DMA,
      ],
      compiler_params=pltpu.CompilerParams(needs_layout_passes=False),
  )


def _make_fin_kernel():
  mesh = plsc.VectorSubcoreMesh(core_axis_name="c", subcore_axis_name="s")

  def body(dots_hbm, inv_hbm, part_hbm, out_hbm, inv_v, d_v, p_v, out_v,
           sem):
    wid = lax.axis_index("s") * NC + lax.axis_index("c")
    base = wid * BW

    pltpu.sync_copy(inv_hbm.at[pl.ds(base, BW)], inv_v)
    pltpu.sync_copy(part_hbm.at[pl.ds(base, BW)], p_v)
    c0 = pltpu.async_copy(dots_hbm.at[inv_v], d_v, sem)
    c0.wait()

    def grp(g, carry):
      gbase = g * L
      out_v[pl.ds(gbase, L)] = d_v[pl.ds(gbase, L)] + p_v[pl.ds(gbase, L)]
      return carry

    lax.fori_loop(0, NG, grp, 0)
    pltpu.sync_copy(out_v, out_hbm.at[pl.ds(base, BW)])

  return pl.kernel(
      body,
      out_type=jax.ShapeDtypeStruct((B,), jnp.float32),
      mesh=mesh,
      scratch_types=[
          pltpu.VMEM((BW,), jnp.int32),
          pltpu.VMEM((BW,), jnp.float32),
          pltpu.VMEM((BW,), jnp.float32),
          pltpu.VMEM((BW,), jnp.float32),
          pltpu.SemaphoreType.DMA,
      ],
      compiler_params=pltpu.CompilerParams(needs_layout_passes=False,
                                           use_tc_tiling_on_sc=False),
  )


@jax.jit
def _mf(user_indices, item_indices, user_embedding, item_embedding,
        user_bias, item_bias):
  u_t = user_embedding.T
  ub = user_bias.reshape(-1)
  ib = item_bias.reshape(-1)

  iot = jnp.arange(B, dtype=jnp.int32)
  su, order = lax.sort((user_indices, iot), num_keys=1)
  inv = jnp.zeros((B,), jnp.int32).at[order].set(iot)
  sbase = lax.shift_right_logical(su, 7) * 128
  s2 = sbase.reshape(NW, BW)
  new = jnp.concatenate(
      [jnp.ones((NW, 1), jnp.bool_), s2[:, 1:] != s2[:, :-1]], axis=1)
  pos = jnp.broadcast_to(jnp.arange(BW, dtype=jnp.int32), (NW, BW))
  keyed = jnp.where(new, pos, 2 * BW)
  rs = jnp.sort(keyed, axis=1)
  rstart = jnp.concatenate(
      [rs, jnp.full((NW, RSW - BW), 2 * BW, jnp.int32)], axis=1)
  nw = jnp.sum(new.astype(jnp.int32), axis=1)

  i_rows, part = _make_item_kernel()(item_indices, user_indices,
                                     item_embedding, ub, ib)
  dots = _make_user_kernel()(su, order, sbase, rstart.reshape(-1), nw,
                             u_t, i_rows)
  return _make_fin_kernel()(dots, inv, part)


def kernel(user_indices, item_indices, user_embedding, item_embedding,
           user_bias, item_bias):
  return _mf(user_indices.astype(jnp.int32), item_indices.astype(jnp.int32),
             user_embedding, item_embedding, user_bias, item_bias)


# speedup vs baseline: 4.5998x; 1.2374x over previous
import functools

import jax
import jax.numpy as jnp
from jax import lax
from jax.experimental import pallas as pl
from jax.experimental.pallas import tpu as pltpu
from jax.experimental.pallas import tpu_sc as plsc

MU2 = 7.0
D = 64
B = 16384
NU = 1000000
L = 16
NC = 2
NS = 16
NW = NC * NS
BW = B // NW
NG = BW // L
RSW = BW + 8
PAD = 16


def _make_item_kernel():
  mesh = plsc.VectorSubcoreMesh(core_axis_name="c", subcore_axis_name="s")

  def body(i_idx_hbm, u_idx_hbm, i_emb_hbm, u_bias_hbm, i_bias_hbm,
           rows_hbm, part_hbm, i_idx_v, u_idx_v, rows_v, ub_v, ib_v,
           part_v, sem):
    wid = lax.axis_index("s") * NC + lax.axis_index("c")
    base = wid * BW

    pltpu.sync_copy(i_idx_hbm.at[pl.ds(base, BW)], i_idx_v)
    pltpu.sync_copy(u_idx_hbm.at[pl.ds(base, BW)], u_idx_v)

    c0 = pltpu.async_copy(i_emb_hbm.at[i_idx_v], rows_v, sem)
    c1 = pltpu.async_copy(u_bias_hbm.at[u_idx_v], ub_v, sem)
    c2 = pltpu.async_copy(i_bias_hbm.at[i_idx_v], ib_v, sem)
    c0.wait()
    c1.wait()
    c2.wait()

    def grp(g, carry):
      gbase = g * L
      part_v[pl.ds(gbase, L)] = (ub_v[pl.ds(gbase, L)] +
                                 ib_v[pl.ds(gbase, L)] + MU2)
      return carry

    lax.fori_loop(0, NG, grp, 0)
    pltpu.sync_copy(rows_v, rows_hbm.at[pl.ds(base, BW), pl.ds(0, D)])
    pltpu.sync_copy(part_v, part_hbm.at[pl.ds(base, BW)])

  return pl.kernel(
      body,
      out_type=(jax.ShapeDtypeStruct((B, 128), jnp.float32),
                jax.ShapeDtypeStruct((B,), jnp.float32)),
      mesh=mesh,
      scratch_types=[
          pltpu.VMEM((BW,), jnp.int32),
          pltpu.VMEM((BW,), jnp.int32),
          pltpu.VMEM((BW, D), jnp.float32),
          pltpu.VMEM((BW,), jnp.float32),
          pltpu.VMEM((BW,), jnp.float32),
          pltpu.VMEM((BW,), jnp.float32),
          pltpu.SemaphoreType.DMA,
      ],
      compiler_params=pltpu.CompilerParams(needs_layout_passes=False,
                                           use_tc_tiling_on_sc=False),
  )


def _make_user_kernel():
  mesh = plsc.VectorSubcoreMesh(core_axis_name="c", subcore_axis_name="s")

  def body(su_hbm, ord_hbm, sb_hbm, rs_hbm, nw_hbm, u_t_hbm, irows_hbm,
           dots_hbm, su_v, ord_v, sb_v, rs_v, nw_v, irows_v, slab_a,
           slab_b, slab_c, slab_d, out_v, sem, sem_a, sem_b, sem_c,
           sem_d):
    wid = lax.axis_index("s") * NC + lax.axis_index("c")
    base = wid * BW

    pltpu.sync_copy(su_hbm.at[pl.ds(base, BW)], su_v.at[pl.ds(0, BW)])
    pltpu.sync_copy(ord_hbm.at[pl.ds(base, BW)], ord_v)
    pltpu.sync_copy(sb_hbm.at[pl.ds(base, BW)], sb_v.at[pl.ds(0, BW)])
    pltpu.sync_copy(rs_hbm.at[pl.ds(wid * RSW, RSW)], rs_v.at[pl.ds(0, RSW)])
    pltpu.sync_copy(nw_hbm, nw_v.at[pl.ds(0, NW)])
    ci = pltpu.async_copy(irows_hbm.at[ord_v], irows_v, sem)

    def sread(ref, i):
      return ref[pl.ds(i, L)][0]

    n = sread(nw_v, wid)

    def fetch(k, buf, fsem):
      @pl.when(k < n)
      def _():
        sb = sread(sb_v, sread(rs_v, k))
        col = pl.multiple_of(sb, 128)
        pltpu.async_copy(u_t_hbm.at[:, pl.ds(col, 128)], buf, fsem)

    fetch(0, slab_a, sem_a)
    fetch(1, slab_b, sem_b)
    fetch(2, slab_c, sem_c)
    fetch(3, slab_d, sem_d)
    ci.wait()

    cvecs = [16 * q + lax.iota(jnp.int32, L) for q in range(4)]

    lane = lax.iota(jnp.int32, L)
    last_lane = lane == (L - 1)

    def run(k, buf):
      rs0 = sread(rs_v, k)
      rs1 = jnp.minimum(sread(rs_v, k + 1), BW)
      sb = sread(sb_v, rs0)

      def elem(e, carry):
        l = sread(su_v, e) - sb
        lv = jnp.full((L,), l, jnp.int32)
        acc = jnp.zeros((L,), jnp.float32)
        for q in range(4):
          uq = plsc.load_gather(buf, [cvecs[q], lv])
          iq = irows_v[e, pl.ds(16 * q, L)]
          acc = acc + uq * iq
        cs = plsc.cumsum(acc)
        plsc.store_scatter(out_v, [jnp.full((L,), e, jnp.int32)], cs,
                           mask=last_lane)
        return carry

      lax.fori_loop(rs0, rs1, elem, 0)

    def step(k, carry):
      parity = lax.rem(k, 4)
      for pv, (buf, fsem) in enumerate([(slab_a, sem_a), (slab_b, sem_b),
                                        (slab_c, sem_c), (slab_d, sem_d)]):
        @pl.when(parity == pv)
        def _(buf=buf, fsem=fsem):
          pltpu.make_async_copy(u_t_hbm.at[:, pl.ds(0, 128)], buf,
                                fsem).wait()
          run(k, buf)
          fetch(k + 4, buf, fsem)

      return carry

    lax.fori_loop(0, n, step, 0)
    pltpu.sync_copy(out_v, dots_hbm.at[pl.ds(base, BW)])

  return pl.kernel(
      body,
      out_type=jax.ShapeDtypeStruct((B,), jnp.float32),
      mesh=mesh,
      scratch_types=[
          pltpu.VMEM((BW + PAD,), jnp.int32),
          pltpu.VMEM((BW,), jnp.int32),
          pltpu.VMEM((BW + PAD,), jnp.int32),
          pltpu.VMEM((RSW + PAD,), jnp.int32),
          pltpu.VMEM((NW + PAD,), jnp.int32),
          pltpu.VMEM((BW, 128), jnp.float32),
          pltpu.VMEM((D, 128), jnp.float32),
          pltpu.VMEM((D, 128), jnp.float32),
          pltpu.VMEM((D, 128), jnp.float32),
          pltpu.VMEM((D, 128), jnp.float32),
          pltpu.VMEM((BW,), jnp.float32),
          pltpu.SemaphoreType.DMA,
          pltpu.SemaphoreType.DMA,
          pltpu.SemaphoreType.DMA,
          pltpu.SemaphoreType.DMA,
          pltpu.SemaphoreType.DMA,
      ],
      compiler_params=pltpu.CompilerParams(needs_layout_passes=False),
  )


def _make_fin_kernel():
  mesh = plsc.VectorSubcoreMesh(core_axis_name="c", subcore_axis_name="s")

  def body(dots_hbm, inv_hbm, part_hbm, out_hbm, inv_v, d_v, p_v, out_v,
           sem):
    wid = lax.axis_index("s") * NC + lax.axis_index("c")
    base = wid * BW

    pltpu.sync_copy(inv_hbm.at[pl.ds(base, BW)], inv_v)
    pltpu.sync_copy(part_hbm.at[pl.ds(base, BW)], p_v)
    c0 = pltpu.async_copy(dots_hbm.at[inv_v], d_v, sem)
    c0.wait()

    def grp(g, carry):
      gbase = g * L
      out_v[pl.ds(gbase, L)] = d_v[pl.ds(gbase, L)] + p_v[pl.ds(gbase, L)]
      return carry

    lax.fori_loop(0, NG, grp, 0)
    pltpu.sync_copy(out_v, out_hbm.at[pl.ds(base, BW)])

  return pl.kernel(
      body,
      out_type=jax.ShapeDtypeStruct((B,), jnp.float32),
      mesh=mesh,
      scratch_types=[
          pltpu.VMEM((BW,), jnp.int32),
          pltpu.VMEM((BW,), jnp.float32),
          pltpu.VMEM((BW,), jnp.float32),
          pltpu.VMEM((BW,), jnp.float32),
          pltpu.SemaphoreType.DMA,
      ],
      compiler_params=pltpu.CompilerParams(needs_layout_passes=False,
                                           use_tc_tiling_on_sc=False),
  )


@jax.jit
def _mf(user_indices, item_indices, user_embedding, item_embedding,
        user_bias, item_bias):
  u_t = user_embedding.T
  ub = user_bias.reshape(-1)
  ib = item_bias.reshape(-1)

  iot = jnp.arange(B, dtype=jnp.int32)
  su, order = lax.sort((user_indices, iot), num_keys=1)
  inv = jnp.zeros((B,), jnp.int32).at[order].set(iot)
  sbase = lax.shift_right_logical(su, 7) * 128
  s2 = sbase.reshape(NW, BW)
  new = jnp.concatenate(
      [jnp.ones((NW, 1), jnp.bool_), s2[:, 1:] != s2[:, :-1]], axis=1)
  pos = jnp.broadcast_to(jnp.arange(BW, dtype=jnp.int32), (NW, BW))
  keyed = jnp.where(new, pos, 2 * BW)
  rs = jnp.sort(keyed, axis=1)
  rstart = jnp.concatenate(
      [rs, jnp.full((NW, RSW - BW), 2 * BW, jnp.int32)], axis=1)
  nw = jnp.sum(new.astype(jnp.int32), axis=1)

  i_rows, part = _make_item_kernel()(item_indices, user_indices,
                                     item_embedding, ub, ib)
  dots = _make_user_kernel()(su, order, sbase, rstart.reshape(-1), nw,
                             u_t, i_rows)
  return _make_fin_kernel()(dots, inv, part)


def kernel(user_indices, item_indices, user_embedding, item_embedding,
           user_bias, item_bias):
  return _mf(user_indices.astype(jnp.int32), item_indices.astype(jnp.int32),
             user_embedding, item_embedding, user_bias, item_bias)


# speedup vs baseline: 4.8536x; 1.0552x over previous
import functools

import jax
import jax.numpy as jnp
from jax import lax
from jax.experimental import pallas as pl
from jax.experimental.pallas import tpu as pltpu
from jax.experimental.pallas import tpu_sc as plsc

MU2 = 7.0
D = 64
B = 16384
NU = 1000000
L = 16
NC = 2
NS = 16
NW = NC * NS
BW = B // NW
NG = BW // L
RSW = BW + 8
PAD = 16


def _make_item_kernel():
  mesh = plsc.VectorSubcoreMesh(core_axis_name="c", subcore_axis_name="s")

  def body(i_idx_hbm, u_idx_hbm, i_emb_hbm, u_bias_hbm, i_bias_hbm,
           rows_hbm, part_hbm, i_idx_v, u_idx_v, rows_v, ub_v, ib_v,
           part_v, sem):
    wid = lax.axis_index("s") * NC + lax.axis_index("c")
    base = wid * BW

    pltpu.sync_copy(i_idx_hbm.at[pl.ds(base, BW)], i_idx_v)
    pltpu.sync_copy(u_idx_hbm.at[pl.ds(base, BW)], u_idx_v)

    c0 = pltpu.async_copy(i_emb_hbm.at[i_idx_v], rows_v, sem)
    c1 = pltpu.async_copy(u_bias_hbm.at[u_idx_v], ub_v, sem)
    c2 = pltpu.async_copy(i_bias_hbm.at[i_idx_v], ib_v, sem)
    c0.wait()
    c1.wait()
    c2.wait()

    def grp(g, carry):
      gbase = g * L
      part_v[pl.ds(gbase, L)] = (ub_v[pl.ds(gbase, L)] +
                                 ib_v[pl.ds(gbase, L)] + MU2)
      return carry

    lax.fori_loop(0, NG, grp, 0)
    pltpu.sync_copy(rows_v, rows_hbm.at[pl.ds(base, BW), pl.ds(0, D)])
    pltpu.sync_copy(part_v, part_hbm.at[pl.ds(base, BW)])

  return pl.kernel(
      body,
      out_type=(jax.ShapeDtypeStruct((B, 128), jnp.float32),
                jax.ShapeDtypeStruct((B,), jnp.float32)),
      mesh=mesh,
      scratch_types=[
          pltpu.VMEM((BW,), jnp.int32),
          pltpu.VMEM((BW,), jnp.int32),
          pltpu.VMEM((BW, D), jnp.float32),
          pltpu.VMEM((BW,), jnp.float32),
          pltpu.VMEM((BW,), jnp.float32),
          pltpu.VMEM((BW,), jnp.float32),
          pltpu.SemaphoreType.DMA,
      ],
      compiler_params=pltpu.CompilerParams(needs_layout_passes=False,
                                           use_tc_tiling_on_sc=False),
  )


def _make_user_kernel():
  mesh = plsc.VectorSubcoreMesh(core_axis_name="c", subcore_axis_name="s")

  def body(su_hbm, ord_hbm, sb_hbm, rs_hbm, nw_hbm, u_t_hbm, irows_hbm,
           dots_hbm, su_v, ord_v, sb_v, rs_v, nw_v, irows_v, slab_a,
           slab_b, slab_c, slab_d, slab_e, slab_f, out_v, sem, sem_a, sem_b,
           sem_c, sem_d, sem_e, sem_f):
    wid = lax.axis_index("s") * NC + lax.axis_index("c")
    base = wid * BW

    pltpu.sync_copy(su_hbm.at[pl.ds(base, BW)], su_v.at[pl.ds(0, BW)])
    pltpu.sync_copy(ord_hbm.at[pl.ds(base, BW)], ord_v)
    pltpu.sync_copy(sb_hbm.at[pl.ds(base, BW)], sb_v.at[pl.ds(0, BW)])
    pltpu.sync_copy(rs_hbm.at[pl.ds(wid * RSW, RSW)], rs_v.at[pl.ds(0, RSW)])
    pltpu.sync_copy(nw_hbm, nw_v.at[pl.ds(0, NW)])
    ci = pltpu.async_copy(irows_hbm.at[ord_v], irows_v, sem)

    def sread(ref, i):
      return ref[pl.ds(i, L)][0]

    n = sread(nw_v, wid)

    def fetch(k, buf, fsem):
      @pl.when(k < n)
      def _():
        sb = sread(sb_v, sread(rs_v, k))
        col = pl.multiple_of(sb, 128)
        pltpu.async_copy(u_t_hbm.at[:, pl.ds(col, 128)], buf, fsem)

    fetch(0, slab_a, sem_a)
    fetch(1, slab_b, sem_b)
    fetch(2, slab_c, sem_c)
    fetch(3, slab_d, sem_d)
    fetch(4, slab_e, sem_e)
    fetch(5, slab_f, sem_f)
    ci.wait()

    cvecs = [16 * q + lax.iota(jnp.int32, L) for q in range(4)]

    lane = lax.iota(jnp.int32, L)
    last_lane = lane == (L - 1)

    def run(k, buf):
      rs0 = sread(rs_v, k)
      rs1 = jnp.minimum(sread(rs_v, k + 1), BW)
      sb = sread(sb_v, rs0)

      def elem(e, carry):
        l = sread(su_v, e) - sb
        lv = jnp.full((L,), l, jnp.int32)
        acc = jnp.zeros((L,), jnp.float32)
        for q in range(4):
          uq = plsc.load_gather(buf, [cvecs[q], lv])
          iq = irows_v[e, pl.ds(16 * q, L)]
          acc = acc + uq * iq
        cs = plsc.cumsum(acc)
        plsc.store_scatter(out_v, [jnp.full((L,), e, jnp.int32)], cs,
                           mask=last_lane)
        return carry

      lax.fori_loop(rs0, rs1, elem, 0)

    def step(k, carry):
      parity = lax.rem(k, 6)
      for pv, (buf, fsem) in enumerate([(slab_a, sem_a), (slab_b, sem_b),
                                        (slab_c, sem_c), (slab_d, sem_d),
                                        (slab_e, sem_e), (slab_f, sem_f)]):
        @pl.when(parity == pv)
        def _(buf=buf, fsem=fsem):
          pltpu.make_async_copy(u_t_hbm.at[:, pl.ds(0, 128)], buf,
                                fsem).wait()
          run(k, buf)
          fetch(k + 6, buf, fsem)

      return carry

    lax.fori_loop(0, n, step, 0)
    pltpu.sync_copy(out_v, dots_hbm.at[pl.ds(base, BW)])

  return pl.kernel(
      body,
      out_type=jax.ShapeDtypeStruct((B,), jnp.float32),
      mesh=mesh,
      scratch_types=[
          pltpu.VMEM((BW + PAD,), jnp.int32),
          pltpu.VMEM((BW,), jnp.int32),
          pltpu.VMEM((BW + PAD,), jnp.int32),
          pltpu.VMEM((RSW + PAD,), jnp.int32),
          pltpu.VMEM((NW + PAD,), jnp.int32),
          pltpu.VMEM((BW, 128), jnp.float32),
          pltpu.VMEM((D, 128), jnp.float32),
          pltpu.VMEM((D, 128), jnp.float32),
          pltpu.VMEM((D, 128), jnp.float32),
          pltpu.VMEM((D, 128), jnp.float32),
          pltpu.VMEM((D, 128), jnp.float32),
          pltpu.VMEM((D, 128), jnp.float32),
          pltpu.VMEM((BW,), jnp.float32),
          pltpu.SemaphoreType.DMA,
          pltpu.SemaphoreType.DMA,
          pltpu.SemaphoreType.DMA,
          pltpu.SemaphoreType.DMA,
          pltpu.SemaphoreType.DMA,
          pltpu.SemaphoreType.DMA,
          pltpu.SemaphoreType.DMA,
      ],
      compiler_params=pltpu.CompilerParams(needs_layout_passes=False),
  )


def _make_fin_kernel():
  mesh = plsc.VectorSubcoreMesh(core_axis_name="c", subcore_axis_name="s")

  def body(dots_hbm, inv_hbm, part_hbm, out_hbm, inv_v, d_v, p_v, out_v,
           sem):
    wid = lax.axis_index("s") * NC + lax.axis_index("c")
    base = wid * BW

    pltpu.sync_copy(inv_hbm.at[pl.ds(base, BW)], inv_v)
    pltpu.sync_copy(part_hbm.at[pl.ds(base, BW)], p_v)
    c0 = pltpu.async_copy(dots_hbm.at[inv_v], d_v, sem)
    c0.wait()

    def grp(g, carry):
      gbase = g * L
      out_v[pl.ds(gbase, L)] = d_v[pl.ds(gbase, L)] + p_v[pl.ds(gbase, L)]
      return carry

    lax.fori_loop(0, NG, grp, 0)
    pltpu.sync_copy(out_v, out_hbm.at[pl.ds(base, BW)])

  return pl.kernel(
      body,
      out_type=jax.ShapeDtypeStruct((B,), jnp.float32),
      mesh=mesh,
      scratch_types=[
          pltpu.VMEM((BW,), jnp.int32),
          pltpu.VMEM((BW,), jnp.float32),
          pltpu.VMEM((BW,), jnp.float32),
          pltpu.VMEM((BW,), jnp.float32),
          pltpu.SemaphoreType.DMA,
      ],
      compiler_params=pltpu.CompilerParams(needs_layout_passes=False,
                                           use_tc_tiling_on_sc=False),
  )


@jax.jit
def _mf(user_indices, item_indices, user_embedding, item_embedding,
        user_bias, item_bias):
  u_t = user_embedding.T
  ub = user_bias.reshape(-1)
  ib = item_bias.reshape(-1)

  iot = jnp.arange(B, dtype=jnp.int32)
  su, order = lax.sort((user_indices, iot), num_keys=1)
  inv = jnp.zeros((B,), jnp.int32).at[order].set(iot)
  sbase = lax.shift_right_logical(su, 7) * 128
  s2 = sbase.reshape(NW, BW)
  new = jnp.concatenate(
      [jnp.ones((NW, 1), jnp.bool_), s2[:, 1:] != s2[:, :-1]], axis=1)
  pos = jnp.broadcast_to(jnp.arange(BW, dtype=jnp.int32), (NW, BW))
  keyed = jnp.where(new, pos, 2 * BW)
  rs = jnp.sort(keyed, axis=1)
  rstart = jnp.concatenate(
      [rs, jnp.full((NW, RSW - BW), 2 * BW, jnp.int32)], axis=1)
  nw = jnp.sum(new.astype(jnp.int32), axis=1)

  i_rows, part = _make_item_kernel()(item_indices, user_indices,
                                     item_embedding, ub, ib)
  dots = _make_user_kernel()(su, order, sbase, rstart.reshape(-1), nw,
                             u_t, i_rows)
  return _make_fin_kernel()(dots, inv, part)


def kernel(user_indices, item_indices, user_embedding, item_embedding,
           user_bias, item_bias):
  return _mf(user_indices.astype(jnp.int32), item_indices.astype(jnp.int32),
             user_embedding, item_embedding, user_bias, item_bias)
